# baseline (speedup 1.0000x reference)
import functools

import jax
import jax.numpy as jnp
from jax import lax
from jax.experimental import pallas as pl
from jax.experimental.pallas import tpu as pltpu
from jax.experimental.pallas import tpu_sc as plsc

E = 8
K = 2
T = 2048
D = 768
F = 1024
BM = 256
NT = (T * K + E * (BM - 1)) // BM
PAD_N = NT * BM

_NW = 32
GN = 6144
_CH = 64
_RPW = GN // _NW
_TPW = T // _NW
_VL = 16


def _gating_body(x_ref, gw_ref, gb_ref, logits_ref, idx_ref, w_ref, cnt_ref):
    x = x_ref[...]
    logits = jnp.dot(x, gw_ref[...].T, preferred_element_type=jnp.float32)
    logits = logits + gb_ref[...]
    logits_ref[...] = logits

    eio = lax.broadcasted_iota(jnp.int32, (T, E), 1)
    m1 = jnp.max(logits, axis=1, keepdims=True)
    i1 = jnp.min(jnp.where(logits == m1, eio, E), axis=1, keepdims=True)
    masked = jnp.where(eio == i1, -jnp.inf, logits)
    m2 = jnp.max(masked, axis=1, keepdims=True)
    i2 = jnp.min(jnp.where(masked == m2, eio, E), axis=1, keepdims=True)

    t = jnp.exp(m2 - m1)
    s = 1.0 + t
    w1 = 1.0 / s
    w2 = t / s

    idx_ref[...] = jnp.concatenate([i1, i2], axis=1).astype(jnp.int32)
    w_ref[...] = jnp.concatenate([w1, w2], axis=1)
    cnt1 = (eio == i1).astype(jnp.float32) + (eio == i2).astype(jnp.float32)
    cnt_ref[...] = jnp.sum(cnt1, axis=0, keepdims=True)


def _gating_call(x2, gate_W, gate_b):
    return pl.pallas_call(
        _gating_body,
        out_shape=[
            jax.ShapeDtypeStruct((T, E), jnp.float32),
            jax.ShapeDtypeStruct((T, K), jnp.int32),
            jax.ShapeDtypeStruct((T, K), jnp.float32),
            jax.ShapeDtypeStruct((1, E), jnp.float32),
        ],
    )(x2, gate_W, gate_b.reshape(1, E))


def _sc_gather_body(
    x_hbm, ids_hbm, out_hbm, idx0, idx1, idx2, buf0, buf1, g0, g1, wsem
):
    wid = lax.axis_index("s") * 2 + lax.axis_index("c")
    base = wid * _RPW
    pltpu.sync_copy(ids_hbm.at[pl.ds(base, _CH)], idx0)
    pltpu.sync_copy(ids_hbm.at[pl.ds(base + _CH, _CH)], idx1)
    pltpu.sync_copy(ids_hbm.at[pl.ds(base + 2 * _CH, _CH)], idx2)
    c0 = pltpu.async_copy(x_hbm.at[idx0], buf0, g0)
    c1 = pltpu.async_copy(x_hbm.at[idx1], buf1, g1)
    c0.wait()
    w0 = pltpu.async_copy(buf0, out_hbm.at[pl.ds(base, _CH)], wsem)
    c1.wait()
    w1 = pltpu.async_copy(buf1, out_hbm.at[pl.ds(base + _CH, _CH)], wsem)
    w0.wait()
    c2 = pltpu.async_copy(x_hbm.at[idx2], buf0, g0)
    c2.wait()
    w2 = pltpu.async_copy(buf0, out_hbm.at[pl.ds(base + 2 * _CH, _CH)], wsem)
    w1.wait()
    w2.wait()


def _sc_gather_call(x2, gather_ids):
    return pl.kernel(
        _sc_gather_body,
        mesh=plsc.VectorSubcoreMesh(core_axis_name="c", subcore_axis_name="s"),
        out_type=jax.ShapeDtypeStruct((GN, D), jnp.float32),
        scratch_types=[
            pltpu.VMEM((_CH,), jnp.int32),
            pltpu.VMEM((_CH,), jnp.int32),
            pltpu.VMEM((_CH,), jnp.int32),
            pltpu.VMEM((_CH, D), jnp.float32),
            pltpu.VMEM((_CH, D), jnp.float32),
            pltpu.SemaphoreType.DMA,
            pltpu.SemaphoreType.DMA,
            pltpu.SemaphoreType.DMA,
        ],
    )(x2, gather_ids)


def _ffn_body(te_ref, xs_ref, w1_ref, b1_ref, w2_ref, b2_ref, ws_ref, ys_ref):
    del te_ref
    xs = xs_ref[...]
    h = jnp.dot(xs, w1_ref[0], preferred_element_type=jnp.float32)
    h = h + b1_ref[0]
    h = 0.5 * h * (1.0 + lax.erf(h * 0.7071067811865476))
    y = jnp.dot(h, w2_ref[0], preferred_element_type=jnp.float32)
    y = y + b2_ref[0]
    ys_ref[...] = y * ws_ref[...]


def _ffn_call(tile_e, xs, W1, b1, W2, b2, ws):
    grid_spec = pltpu.PrefetchScalarGridSpec(
        num_scalar_prefetch=1,
        grid=(NT,),
        in_specs=[
            pl.BlockSpec((BM, D), lambda i, te: (i, 0)),
            pl.BlockSpec((1, D, F), lambda i, te: (te[i], 0, 0)),
            pl.BlockSpec((1, 1, F), lambda i, te: (te[i], 0, 0)),
            pl.BlockSpec((1, F, D), lambda i, te: (te[i], 0, 0)),
            pl.BlockSpec((1, 1, D), lambda i, te: (te[i], 0, 0)),
            pl.BlockSpec((BM, 1), lambda i, te: (i, 0)),
        ],
        out_specs=pl.BlockSpec((BM, D), lambda i, te: (i, 0)),
    )
    return pl.pallas_call(
        _ffn_body,
        grid_spec=grid_spec,
        out_shape=jax.ShapeDtypeStruct((PAD_N, D), jnp.float32),
    )(tile_e, xs, W1, b1.reshape(E, 1, F), W2, b2.reshape(E, 1, D), ws)


def _sc_combine_body(
    ys_hbm, p0_hbm, p1_hbm, out_hbm, i0_v, i1_v, r0_v, r1_v, sem0, sem1
):
    wid = lax.axis_index("s") * 2 + lax.axis_index("c")
    base = wid * _TPW
    pltpu.sync_copy(p0_hbm.at[pl.ds(base, _TPW)], i0_v)
    pltpu.sync_copy(p1_hbm.at[pl.ds(base, _TPW)], i1_v)
    c0 = pltpu.async_copy(ys_hbm.at[i0_v], r0_v, sem0)
    c1 = pltpu.async_copy(ys_hbm.at[i1_v], r1_v, sem1)
    c0.wait()
    c1.wait()

    def add_row(i, _):
        for j in range(D // _VL):
            sl = pl.ds(j * _VL, _VL)
            r0_v[i, sl] = r0_v[i, sl] + r1_v[i, sl]
        return 0

    lax.fori_loop(0, _TPW, add_row, 0)
    pltpu.sync_copy(r0_v, out_hbm.at[pl.ds(base, _TPW)])


def _sc_combine_call(ys, p0, p1):
    return pl.kernel(
        _sc_combine_body,
        mesh=plsc.VectorSubcoreMesh(core_axis_name="c", subcore_axis_name="s"),
        out_type=jax.ShapeDtypeStruct((T, D), jnp.float32),
        scratch_types=[
            pltpu.VMEM((_TPW,), jnp.int32),
            pltpu.VMEM((_TPW,), jnp.int32),
            pltpu.VMEM((_TPW, D), jnp.float32),
            pltpu.VMEM((_TPW, D), jnp.float32),
            pltpu.SemaphoreType.DMA,
            pltpu.SemaphoreType.DMA,
        ],
    )(ys, p0, p1)


def kernel(x, gate_W, gate_b, W1, b1, W2, b2):
    x2 = x.reshape(T, D)
    logits, idx2, w2, cnt = _gating_call(x2, gate_W, gate_b)

    flat_e = idx2.reshape(-1)
    onehot = (flat_e[:, None] == jnp.arange(E, dtype=jnp.int32)[None, :])
    rank = jnp.take_along_axis(
        jnp.cumsum(onehot.astype(jnp.int32), axis=0), flat_e[:, None], axis=1
    )[:, 0] - 1
    counts_i = cnt.reshape(E).astype(jnp.int32)
    acnt = ((counts_i + BM - 1) // BM) * BM
    aend = jnp.cumsum(acnt)
    aoff = aend - acnt
    pos = aoff[flat_e] + rank

    tokens = jnp.arange(T * K, dtype=jnp.int32) // K
    gather_ids = jnp.zeros((GN,), jnp.int32).at[pos].set(tokens)
    ws = jnp.zeros((PAD_N,), jnp.float32).at[pos].set(w2.reshape(-1))
    tile_e = jnp.minimum(
        jnp.searchsorted(aend, jnp.arange(NT, dtype=jnp.int32) * BM, side="right"),
        E - 1,
    ).astype(jnp.int32)
    p0 = pos[0::2]
    p1 = pos[1::2]

    xs = _sc_gather_call(x2, gather_ids)
    ys = _ffn_call(tile_e, xs, W1, b1, W2, b2, ws.reshape(PAD_N, 1))
    out = _sc_combine_call(ys, p0, p1)

    return (
        out.reshape(1, T, D),
        logits.reshape(1, T, E),
        idx2.reshape(1, T, K),
        cnt.reshape(E),
    )


# speedup vs baseline: 1.6263x; 1.6263x over previous
import functools

import jax
import jax.numpy as jnp
from jax import lax
from jax.experimental import pallas as pl
from jax.experimental.pallas import tpu as pltpu
from jax.experimental.pallas import tpu_sc as plsc

E = 8
K = 2
T = 2048
D = 768
F = 1024
BM = 256
NT = (T * K + E * (BM - 1)) // BM
PAD_N = NT * BM

_NW = 32
GN = 6144
_CH = 64
_RPW = GN // _NW
_TPW = T // _NW
_VL = 16


def _gating_body(x_ref, gw_ref, gb_ref, logits_ref, idx_ref, w_ref, cnt_ref):
    x = x_ref[...]
    logits = jnp.dot(x, gw_ref[...].T, preferred_element_type=jnp.float32)
    logits = logits + gb_ref[...]
    logits_ref[...] = logits

    eio = lax.broadcasted_iota(jnp.int32, (T, E), 1)
    m1 = jnp.max(logits, axis=1, keepdims=True)
    i1 = jnp.min(jnp.where(logits == m1, eio, E), axis=1, keepdims=True)
    masked = jnp.where(eio == i1, -jnp.inf, logits)
    m2 = jnp.max(masked, axis=1, keepdims=True)
    i2 = jnp.min(jnp.where(masked == m2, eio, E), axis=1, keepdims=True)

    t = jnp.exp(m2 - m1)
    s = 1.0 + t
    w1 = 1.0 / s
    w2 = t / s

    idx_ref[...] = jnp.concatenate([i1, i2], axis=1).astype(jnp.int32)
    w_ref[...] = jnp.concatenate([w1, w2], axis=1)
    cnt1 = (eio == i1).astype(jnp.float32) + (eio == i2).astype(jnp.float32)
    cnt_ref[...] = jnp.sum(cnt1, axis=0, keepdims=True)


def _gating_call(x2, gate_W, gate_b):
    return pl.pallas_call(
        _gating_body,
        out_shape=[
            jax.ShapeDtypeStruct((T, E), jnp.float32),
            jax.ShapeDtypeStruct((T, K), jnp.int32),
            jax.ShapeDtypeStruct((T, K), jnp.float32),
            jax.ShapeDtypeStruct((1, E), jnp.float32),
        ],
    )(x2, gate_W, gate_b.reshape(1, E))


def _sc_gather_body(
    x_hbm, ids_hbm, out_hbm, idx0, idx1, idx2, buf0, buf1, g0, g1, wsem
):
    wid = lax.axis_index("s") * 2 + lax.axis_index("c")
    base = wid * _RPW
    pltpu.sync_copy(ids_hbm.at[pl.ds(base, _CH)], idx0)
    pltpu.sync_copy(ids_hbm.at[pl.ds(base + _CH, _CH)], idx1)
    pltpu.sync_copy(ids_hbm.at[pl.ds(base + 2 * _CH, _CH)], idx2)
    c0 = pltpu.async_copy(x_hbm.at[idx0], buf0, g0)
    c1 = pltpu.async_copy(x_hbm.at[idx1], buf1, g1)
    c0.wait()
    w0 = pltpu.async_copy(buf0, out_hbm.at[pl.ds(base, _CH)], wsem)
    c1.wait()
    w1 = pltpu.async_copy(buf1, out_hbm.at[pl.ds(base + _CH, _CH)], wsem)
    w0.wait()
    c2 = pltpu.async_copy(x_hbm.at[idx2], buf0, g0)
    c2.wait()
    w2 = pltpu.async_copy(buf0, out_hbm.at[pl.ds(base + 2 * _CH, _CH)], wsem)
    w1.wait()
    w2.wait()


def _sc_gather_call(x2, gather_ids):
    return pl.kernel(
        _sc_gather_body,
        mesh=plsc.VectorSubcoreMesh(core_axis_name="c", subcore_axis_name="s"),
        out_type=jax.ShapeDtypeStruct((GN, D), jnp.float32),
        scratch_types=[
            pltpu.VMEM((_CH,), jnp.int32),
            pltpu.VMEM((_CH,), jnp.int32),
            pltpu.VMEM((_CH,), jnp.int32),
            pltpu.VMEM((_CH, D), jnp.float32),
            pltpu.VMEM((_CH, D), jnp.float32),
            pltpu.SemaphoreType.DMA,
            pltpu.SemaphoreType.DMA,
            pltpu.SemaphoreType.DMA,
        ],
    )(x2, gather_ids)


def _ffn_body(te_ref, xs_ref, w1_ref, b1_ref, w2_ref, b2_ref, ws_ref, ys_ref):
    del te_ref
    xs = xs_ref[...]
    h = jnp.dot(xs, w1_ref[0], preferred_element_type=jnp.float32)
    h = h + b1_ref[0]
    h = 0.5 * h * (1.0 + lax.erf(h * 0.7071067811865476))
    y = jnp.dot(h, w2_ref[0], preferred_element_type=jnp.float32)
    y = y + b2_ref[0]
    ys_ref[...] = y * ws_ref[...]


def _ffn_call(tile_e, xs, W1, b1, W2, b2, ws):
    grid_spec = pltpu.PrefetchScalarGridSpec(
        num_scalar_prefetch=1,
        grid=(NT,),
        in_specs=[
            pl.BlockSpec((BM, D), lambda i, te: (i, 0)),
            pl.BlockSpec((1, D, F), lambda i, te: (te[i], 0, 0)),
            pl.BlockSpec((1, 1, F), lambda i, te: (te[i], 0, 0)),
            pl.BlockSpec((1, F, D), lambda i, te: (te[i], 0, 0)),
            pl.BlockSpec((1, 1, D), lambda i, te: (te[i], 0, 0)),
            pl.BlockSpec((BM, 1), lambda i, te: (i, 0)),
        ],
        out_specs=pl.BlockSpec((BM, D), lambda i, te: (i, 0)),
    )
    return pl.pallas_call(
        _ffn_body,
        grid_spec=grid_spec,
        out_shape=jax.ShapeDtypeStruct((PAD_N, D), jnp.float32),
    )(tile_e, xs, W1, b1.reshape(E, 1, F), W2, b2.reshape(E, 1, D), ws)


def _sc_combine_body(
    ys_hbm, p0_hbm, p1_hbm, out_hbm, i0_v, i1_v, r0_v, r1_v, sem0, sem1
):
    wid = lax.axis_index("s") * 2 + lax.axis_index("c")
    base = wid * _TPW
    pltpu.sync_copy(p0_hbm.at[pl.ds(base, _TPW)], i0_v)
    pltpu.sync_copy(p1_hbm.at[pl.ds(base, _TPW)], i1_v)
    c0 = pltpu.async_copy(ys_hbm.at[i0_v], r0_v, sem0)
    c1 = pltpu.async_copy(ys_hbm.at[i1_v], r1_v, sem1)
    c0.wait()
    c1.wait()

    def add_row(i, _):
        for j in range(D // _VL):
            sl = pl.ds(j * _VL, _VL)
            r0_v[i, sl] = r0_v[i, sl] + r1_v[i, sl]
        return 0

    lax.fori_loop(0, _TPW, add_row, 0)
    pltpu.sync_copy(r0_v, out_hbm.at[pl.ds(base, _TPW)])


def _sc_combine_call(ys, p0, p1):
    return pl.kernel(
        _sc_combine_body,
        mesh=plsc.VectorSubcoreMesh(core_axis_name="c", subcore_axis_name="s"),
        out_type=jax.ShapeDtypeStruct((T, D), jnp.float32),
        scratch_types=[
            pltpu.VMEM((_TPW,), jnp.int32),
            pltpu.VMEM((_TPW,), jnp.int32),
            pltpu.VMEM((_TPW, D), jnp.float32),
            pltpu.VMEM((_TPW, D), jnp.float32),
            pltpu.SemaphoreType.DMA,
            pltpu.SemaphoreType.DMA,
        ],
    )(ys, p0, p1)


def kernel(x, gate_W, gate_b, W1, b1, W2, b2):
    x2 = x.reshape(T, D)
    logits, idx2, w2, cnt = _gating_call(x2, gate_W, gate_b)

    flat_e = idx2.reshape(-1)
    onehot = (flat_e[:, None] == jnp.arange(E, dtype=jnp.int32)[None, :])
    rank = jnp.take_along_axis(
        jnp.cumsum(onehot.astype(jnp.int32), axis=0), flat_e[:, None], axis=1
    )[:, 0] - 1
    counts_i = cnt.reshape(E).astype(jnp.int32)
    acnt = ((counts_i + BM - 1) // BM) * BM
    aend = jnp.cumsum(acnt)
    aoff = aend - acnt
    pos = aoff[flat_e] + rank

    tokens = jnp.arange(T * K, dtype=jnp.int32) // K
    gather_ids = (jnp.arange(GN, dtype=jnp.int32) % T).at[pos].set(tokens)
    ws = jnp.zeros((PAD_N,), jnp.float32).at[pos].set(w2.reshape(-1))
    tile_e = jnp.minimum(
        jnp.searchsorted(aend, jnp.arange(NT, dtype=jnp.int32) * BM, side="right"),
        E - 1,
    ).astype(jnp.int32)
    p0 = pos[0::2]
    p1 = pos[1::2]

    xs = _sc_gather_call(x2, gather_ids)
    ys = _ffn_call(tile_e, xs, W1, b1, W2, b2, ws.reshape(PAD_N, 1))
    out = _sc_combine_call(ys, p0, p1)

    return (
        out.reshape(1, T, D),
        logits.reshape(1, T, E),
        idx2.reshape(1, T, K),
        cnt.reshape(E),
    )
